# baseline (speedup 1.0000x reference)
import jax
import jax.numpy as jnp
from jax.experimental import pallas as pl
from jax.experimental.pallas import tpu as pltpu

_CHUNK = 2048


def _chunks(n):
    return [(o, min(_CHUNK, n - o)) for o in range(0, n, _CHUNK)]


def _mlp_body(x_hbm, w1t_hbm, b1_hbm, w2t_hbm, b2_hbm, o_hbm, *scratch):
    n = x_hbm.shape[0]
    chunks = _chunks(n)
    k = len(chunks)
    xvs = scratch[:k]
    ovs = scratch[k:2 * k]
    w1tv, b1v, w2tv, b2v, lsem, wsem, ssem = scratch[2 * k:]

    wcopies = (
        pltpu.make_async_copy(w1t_hbm, w1tv, wsem.at[0]),
        pltpu.make_async_copy(b1_hbm, b1v, wsem.at[1]),
        pltpu.make_async_copy(w2t_hbm, w2tv, wsem.at[2]),
        pltpu.make_async_copy(b2_hbm, b2v, wsem.at[3]),
    )
    for c in wcopies:
        c.start()

    lcopies = []
    for i, (o, s) in enumerate(chunks):
        c = pltpu.make_async_copy(
            x_hbm.at[pl.ds(o, s), :], xvs[i], lsem.at[i]
        )
        c.start()
        lcopies.append(c)

    for c in wcopies:
        c.wait()
    b1c = b1v[:][None, :].T
    b2c = b2v[:][None, :].T

    scopies = []
    for i, (o, s) in enumerate(chunks):
        lcopies[i].wait()
        hT = jax.lax.dot_general(
            w1tv[:], xvs[i][:], (((1,), (1,)), ((), ())),
            preferred_element_type=jnp.float32,
        )
        hT = jnp.maximum(hT + b1c, 0.0)
        ovs[i][:] = (
            jnp.dot(w2tv[:], hT, preferred_element_type=jnp.float32) + b2c
        )
        c = pltpu.make_async_copy(
            ovs[i], o_hbm.at[:, pl.ds(o, s)], ssem.at[i]
        )
        c.start()
        scopies.append(c)

    for c in scopies:
        c.wait()


def kernel(x, edge_index, W1, b1, W2, b2):
    n, in_ch = x.shape
    hid = W1.shape[1]
    ncls = W2.shape[1]
    k = len(_chunks(n))
    outT = pl.pallas_call(
        _mlp_body,
        in_specs=[pl.BlockSpec(memory_space=pltpu.MemorySpace.HBM)] * 5,
        out_specs=pl.BlockSpec(memory_space=pltpu.MemorySpace.HBM),
        out_shape=jax.ShapeDtypeStruct((ncls, n), jnp.float32),
        scratch_shapes=(
            [pltpu.VMEM((s, in_ch), jnp.float32) for _, s in _chunks(n)]
            + [pltpu.VMEM((ncls, s), jnp.float32) for _, s in _chunks(n)]
            + [
                pltpu.VMEM((hid, in_ch), jnp.float32),
                pltpu.VMEM((hid,), jnp.float32),
                pltpu.VMEM((ncls, hid), jnp.float32),
                pltpu.VMEM((ncls,), jnp.float32),
                pltpu.SemaphoreType.DMA((k,)),
                pltpu.SemaphoreType.DMA((4,)),
                pltpu.SemaphoreType.DMA((k,)),
            ]
        ),
    )(x, W1.T, b1, W2.T, b2)
    return outT.T
